# Initial kernel scaffold; baseline (speedup 1.0000x reference)
import jax
import jax.numpy as jnp
from jax.experimental import pallas as pl


def kernel(q_pts, s_pts, neighb_inds, x, K_points, W):
    raise NotImplementedError("write your pallas kernel here")


# speedup vs baseline: 1.2087x; 1.2087x over previous
import functools

import jax
import jax.numpy as jnp
from jax import lax
from jax.experimental import pallas as pl
from jax.experimental.pallas import tpu as pltpu
from jax.experimental.pallas import tpu_sc as plsc

N = 10000
H = 32
K = 15
CIN = 128
COUT = 128
KP_EXTENT = 1.2
E = N * H

NC = 2
NS = 16
NW = NC * NS

CH = 128
NCHUNK = E // CH
MAXC = (NCHUNK + NW - 1) // NW

B = 200
BH = B * H
GB = N // B


def _sc_gather_body(table_hbm, inds_hbm, xn_hbm, idx_v, rows_v, sem):
    wid = lax.axis_index("s") * NC + lax.axis_index("c")

    def body(i, carry):
        c = wid + i * NW

        @pl.when(c < NCHUNK)
        def _():
            off = pl.multiple_of(c * CH, CH)
            pltpu.sync_copy(inds_hbm.at[pl.ds(off, CH)], idx_v)
            pltpu.async_copy(table_hbm.at[idx_v], rows_v, sem).wait()
            pltpu.sync_copy(rows_v, xn_hbm.at[pl.ds(off, CH)])

        return carry

    lax.fori_loop(0, MAXC, body, 0)


def _sc_gather(table, inds):
    mesh = plsc.VectorSubcoreMesh(core_axis_name="c", subcore_axis_name="s")
    fn = pl.kernel(
        _sc_gather_body,
        mesh=mesh,
        out_type=jax.ShapeDtypeStruct((E, CIN), jnp.float32),
        scratch_types=[
            pltpu.VMEM((CH,), jnp.int32),
            pltpu.VMEM((CH, CIN), jnp.float32),
            pltpu.SemaphoreType.DMA,
        ],
    )
    return fn(table, inds)


def _tc_body(q_ref, kt_ref, k2_ref, w_ref, xn_ref, out_ref):
    raw = xn_ref[...]
    wi = lax.bitcast_convert_type(raw[:, 0:64], jnp.int32)
    f_even = lax.bitcast_convert_type(
        wi & jnp.int32(-65536), jnp.float32)
    f_odd = lax.bitcast_convert_type(wi << 16, jnp.float32)
    feats = jnp.concatenate([f_even, f_odd], axis=1)
    c3 = raw[:, 64:67]
    q = q_ref[...]
    qb = jnp.broadcast_to(q[:, None, :], (B, H, 3)).reshape(BH, 3)
    n3 = c3 - qb
    dots = jnp.dot(n3, kt_ref[...], preferred_element_type=jnp.float32)
    n2 = jnp.sum(n3 * n3, axis=1, keepdims=True)
    sq = jnp.maximum(n2 - 2.0 * dots + k2_ref[...], 0.0)
    wgt = jnp.maximum(1.0 - jnp.sqrt(sq) * (1.0 / KP_EXTENT), 0.0)
    parts = []
    for k in range(K):
        g = feats * wgt[:, k:k + 1]
        parts.append(jnp.sum(g.reshape(B, H, CIN), axis=1))
    a = jnp.concatenate(parts, axis=1)
    out_ref[...] = jnp.dot(a, w_ref[...], preferred_element_type=jnp.float32)


def _tc_call(q_pts, kt, k2, wflat, xn):
    return pl.pallas_call(
        _tc_body,
        grid=(GB,),
        in_specs=[
            pl.BlockSpec((B, 3), lambda i: (i, 0)),
            pl.BlockSpec((3, K), lambda i: (0, 0)),
            pl.BlockSpec((1, K), lambda i: (0, 0)),
            pl.BlockSpec((K * CIN, COUT), lambda i: (0, 0)),
            pl.BlockSpec((BH, CIN), lambda i: (i, 0)),
        ],
        out_specs=pl.BlockSpec((B, COUT), lambda i: (i, 0)),
        out_shape=jax.ShapeDtypeStruct((N, COUT), jnp.float32),
    )(q_pts, kt, k2, wflat, xn)


def _pack_table(x, s_pts):
    xb = x.astype(jnp.bfloat16)
    u = lax.bitcast_convert_type(xb, jnp.uint16).astype(jnp.uint32)
    w = (u[:, 0::2] << 16) | u[:, 1::2]
    packedf = lax.bitcast_convert_type(w, jnp.float32)
    row = jnp.concatenate(
        [packedf, s_pts, jnp.zeros((N, 61), jnp.float32)], axis=1)
    shadow = jnp.zeros((1, 128), jnp.float32).at[0, 64:67].set(1e6)
    return jnp.concatenate([row, shadow], axis=0)


def kernel(q_pts, s_pts, neighb_inds, x, K_points, W):
    table = _pack_table(x, s_pts)
    inds = neighb_inds.astype(jnp.int32).reshape(E)
    xn = _sc_gather(table, inds)
    kt = K_points.T.astype(jnp.float32)
    k2 = jnp.sum(K_points * K_points, axis=1)[None, :]
    perm = jnp.arange(CIN).reshape(64, 2).T.reshape(CIN)
    wflat = W[:, perm, :].reshape(K * CIN, COUT)
    return _tc_call(q_pts, kt, k2, wflat, xn)


# speedup vs baseline: 2.2275x; 1.8429x over previous
import functools

import jax
import jax.numpy as jnp
from jax import lax
from jax.experimental import pallas as pl
from jax.experimental.pallas import tpu as pltpu
from jax.experimental.pallas import tpu_sc as plsc

N = 10000
H = 32
K = 15
CIN = 128
COUT = 128
KP_EXTENT = 1.2
E = N * H

NC = 2
NS = 16
NW = NC * NS

CH = 128
NCHUNK = E // CH
MAXC = (NCHUNK + NW - 1) // NW

B = 200
BH = B * H
GB = N // B


def _sc_gather_body(table_hbm, inds_hbm, xn_hbm, idx_v, rows_v, sem):
    wid = lax.axis_index("s") * NC + lax.axis_index("c")

    def body(i, carry):
        c = wid + i * NW

        @pl.when(c < NCHUNK)
        def _():
            off = pl.multiple_of(c * CH, CH)
            pltpu.sync_copy(inds_hbm.at[pl.ds(off, CH)], idx_v)
            pltpu.async_copy(table_hbm.at[idx_v], rows_v, sem).wait()
            pltpu.sync_copy(rows_v, xn_hbm.at[pl.ds(off, CH)])

        return carry

    lax.fori_loop(0, MAXC, body, 0)


def _sc_gather(table, inds):
    mesh = plsc.VectorSubcoreMesh(core_axis_name="c", subcore_axis_name="s")
    fn = pl.kernel(
        _sc_gather_body,
        mesh=mesh,
        out_type=jax.ShapeDtypeStruct((E, CIN), jnp.float32),
        scratch_types=[
            pltpu.VMEM((CH,), jnp.int32),
            pltpu.VMEM((CH, CIN), jnp.float32),
            pltpu.SemaphoreType.DMA,
        ],
    )
    return fn(table, inds)


def _tc_body(q_ref, kt_ref, k2_ref, w_ref, xn_ref, out_ref):
    raw = xn_ref[...]
    wi = lax.bitcast_convert_type(raw[:, 0:64], jnp.int32)
    f_even = lax.bitcast_convert_type(
        wi & jnp.int32(-65536), jnp.float32)
    f_odd = lax.bitcast_convert_type(wi << 16, jnp.float32)
    feats = jnp.concatenate([f_even, f_odd], axis=1)
    c3 = raw[:, 64:67]
    q = q_ref[...]
    qb = jnp.broadcast_to(q[:, None, :], (B, H, 3)).reshape(BH, 3)
    n3 = c3 - qb
    dots = jnp.dot(n3, kt_ref[...], preferred_element_type=jnp.float32)
    n2 = jnp.sum(n3 * n3, axis=1, keepdims=True)
    sq = jnp.maximum(n2 - 2.0 * dots + k2_ref[...], 0.0)
    wgt = jnp.maximum(1.0 - jnp.sqrt(sq) * (1.0 / KP_EXTENT), 0.0)
    w3 = wgt.reshape(B, H, K)
    f3 = feats.reshape(B, H, CIN)
    a3 = lax.dot_general(w3, f3, (((1,), (1,)), ((0,), (0,))),
                         preferred_element_type=jnp.float32)
    a = a3.reshape(B, K * CIN)
    out_ref[...] = jnp.dot(a, w_ref[...], preferred_element_type=jnp.float32)


def _tc_call(q_pts, kt, k2, wflat, xn):
    return pl.pallas_call(
        _tc_body,
        grid=(GB,),
        in_specs=[
            pl.BlockSpec((B, 3), lambda i: (i, 0)),
            pl.BlockSpec((3, K), lambda i: (0, 0)),
            pl.BlockSpec((1, K), lambda i: (0, 0)),
            pl.BlockSpec((K * CIN, COUT), lambda i: (0, 0)),
            pl.BlockSpec((BH, CIN), lambda i: (i, 0)),
        ],
        out_specs=pl.BlockSpec((B, COUT), lambda i: (i, 0)),
        out_shape=jax.ShapeDtypeStruct((N, COUT), jnp.float32),
    )(q_pts, kt, k2, wflat, xn)


def _pack_table(x, s_pts):
    xb = x.astype(jnp.bfloat16)
    u = lax.bitcast_convert_type(xb, jnp.uint16).astype(jnp.uint32)
    w = (u[:, 0::2] << 16) | u[:, 1::2]
    packedf = lax.bitcast_convert_type(w, jnp.float32)
    row = jnp.concatenate(
        [packedf, s_pts, jnp.zeros((N, 61), jnp.float32)], axis=1)
    shadow = jnp.zeros((1, 128), jnp.float32).at[0, 64:67].set(1e6)
    return jnp.concatenate([row, shadow], axis=0)


def kernel(q_pts, s_pts, neighb_inds, x, K_points, W):
    table = _pack_table(x, s_pts)
    inds = neighb_inds.astype(jnp.int32).reshape(E)
    xn = _sc_gather(table, inds)
    kt = K_points.T.astype(jnp.float32)
    k2 = jnp.sum(K_points * K_points, axis=1)[None, :]
    perm = jnp.arange(CIN).reshape(64, 2).T.reshape(CIN)
    wflat = W[:, perm, :].reshape(K * CIN, COUT)
    return _tc_call(q_pts, kt, k2, wflat, xn)
